# Initial kernel scaffold; baseline (speedup 1.0000x reference)
import jax
import jax.numpy as jnp
from jax.experimental import pallas as pl


def kernel(x, edge_index, W1, a1_src, a1_dst, b1, W2, a2_src, a2_dst, b2, fcg_w, fcg_b, fc1_w, fc1_b, fc2_w, fc2_b, out_w, out_b):
    raise NotImplementedError("write your pallas kernel here")


# speedup vs baseline: 1.0834x; 1.0834x over previous
import functools

import jax
import jax.numpy as jnp
from jax.experimental import pallas as pl
from jax.experimental.pallas import tpu as pltpu

N = 10000
E = 160000
F = 128
HEADS = 10
NPAD = 10240
EPAD = 172032


def _matmul_kernel(x_ref, w_ref, o_ref):
    o_ref[...] = jnp.dot(x_ref[...], w_ref[...],
                         preferred_element_type=jnp.float32)


def _matmul(x, w, bm=256):
    m, k = x.shape
    k2, n = w.shape
    grid = (m // bm,)
    return pl.pallas_call(
        _matmul_kernel,
        grid=grid,
        in_specs=[
            pl.BlockSpec((bm, k), lambda i: (i, 0)),
            pl.BlockSpec((k, n), lambda i: (0, 0)),
        ],
        out_specs=pl.BlockSpec((bm, n), lambda i: (i, 0)),
        out_shape=jax.ShapeDtypeStruct((m, n), jnp.float32),
    )(x, w)


def _gat_xla(h, src, dst, alpha_s, alpha_d, n, heads, out_ch):
    e = alpha_s[src] + alpha_d[dst]
    e = jnp.where(e >= 0, e, 0.2 * e)
    w = jnp.exp(e)
    denom = jax.ops.segment_sum(w, dst, num_segments=n)
    hr = h.reshape(-1, heads, out_ch)
    msg = hr[src] * w[:, :, None]
    acc = jax.ops.segment_sum(msg, dst, num_segments=n)
    return acc / (denom[:, :, None] + 1e-16)


def kernel(x, edge_index, W1, a1_src, a1_dst, b1, W2, a2_src, a2_dst, b2,
           fcg_w, fcg_b, fc1_w, fc1_b, fc2_w, fc2_b, out_w, out_b):
    loop = jnp.arange(N, dtype=edge_index.dtype)
    src = jnp.concatenate([edge_index[0], loop])
    dst = jnp.concatenate([edge_index[1], loop])

    xp = jnp.pad(x, ((0, NPAD - N), (0, 0)))

    h1 = _matmul(xp, W1)
    h1r = h1.reshape(NPAD, HEADS, F)
    a1s = jnp.sum(h1r * a1_src[None], axis=-1)
    a1d = jnp.sum(h1r * a1_dst[None], axis=-1)
    g1 = _gat_xla(h1[:N], src, dst, a1s[:N], a1d[:N], N, HEADS, F)
    hin = jax.nn.elu(g1.reshape(N, HEADS * F) + b1)

    hinp = jnp.pad(hin, ((0, NPAD - N), (0, 0)))
    h2 = _matmul(hinp, W2)
    a2s = h2 @ a2_src.T
    a2d = h2 @ a2_dst.T
    g2 = _gat_xla(h2[:N], src, dst, a2s[:N], a2d[:N], N, 1, 128)
    h_out = jax.nn.relu(g2.reshape(N, 128) + b2)

    g = jnp.max(h_out, axis=0, keepdims=True)
    g = jax.nn.relu(g @ fcg_w + fcg_b)
    g = jax.nn.relu(g @ fc1_w + fc1_b)
    g = jax.nn.relu(g @ fc2_w + fc2_b)
    return g @ out_w + out_b


# speedup vs baseline: 8.0329x; 7.4143x over previous
import functools

import jax
import jax.numpy as jnp
from jax import lax
from jax.experimental import pallas as pl
from jax.experimental.pallas import tpu as pltpu
from jax.experimental.pallas import tpu_sc as plsc

N = 10000
E = 160000
F = 128
HEADS = 10
NPAD = 10240
NC = 2
NS = 16
NTILES = NC * NS
BB = 128
NB = 42
EPAD = NTILES * NB * BB
FH = 64
FW = 80
BM = 256
RPT = NPAD // NS
NZ = RPT // BB


def _make_edge_sc(H):
    mesh = plsc.VectorSubcoreMesh(core_axis_name="c", subcore_axis_name="s")

    @functools.partial(
        pl.kernel,
        out_type=(
            jax.ShapeDtypeStruct((NC, 2 * H, NPAD, FW), jnp.float32),
            jax.ShapeDtypeStruct((NTILES, NB, BB, 16), jnp.float32),
        ),
        mesh=mesh,
        compiler_params=pltpu.CompilerParams(use_tc_tiling_on_sc=False),
        scratch_types=[
            pltpu.VMEM((NB, BB), jnp.int32),
            pltpu.VMEM((NB, BB), jnp.int32),
            pltpu.VMEM((BB, 16), jnp.float32),
            pltpu.VMEM((BB, 16), jnp.float32),
            pltpu.VMEM((BB, 16), jnp.float32),
            pltpu.VMEM((BB, FW), jnp.float32),
            pltpu.VMEM((BB, FW), jnp.float32),
            pltpu.VMEM_SHARED((NPAD, FW), jnp.float32),
            pltpu.SemaphoreType.DMA,
            pltpu.SemaphoreType.DMA,
        ],
    )
    def edge_kernel(feat, a_s, a_d, srcs, dsts, parts, wout,
                    idx_s, idx_d, ar, ad, wb, rows, zbuf, acc, sem1, sem2):
        cid = lax.axis_index("c")
        sid = lax.axis_index("s")
        tid = cid * NS + sid
        fzero = jnp.zeros((16,), jnp.float32)

        def zb(j, _):
            for k2 in range(FW // 16):
                zbuf[j, k2 * 16:(k2 + 1) * 16] = fzero
            return 0
        lax.fori_loop(0, BB, zb, 0)

        pltpu.sync_copy(srcs.at[tid], idx_s)
        pltpu.sync_copy(dsts.at[tid], idx_d)

        def batch_a(b, _):
            ga = pltpu.async_copy(a_s.at[idx_s.at[b]], ar, sem1)
            gb = pltpu.async_copy(a_d.at[idx_d.at[b]], ad, sem2)
            ga.wait()
            gb.wait()

            def row_a(j, _):
                e = ar[j, :] + ad[j, :]
                e = jnp.where(e >= 0.0, e, e * 0.2)
                wb[j, :] = jnp.exp(e)
                return 0
            lax.fori_loop(0, BB, row_a, 0)
            pltpu.sync_copy(wb, wout.at[tid, b])
            return 0
        lax.fori_loop(0, NB, batch_a, 0)

        def head(hh, _):
            def zc(i, _):
                pltpu.sync_copy(zbuf, acc.at[pl.ds(sid * RPT + i * BB, BB)])
                return 0
            lax.fori_loop(0, NZ, zc, 0)
            plsc.subcore_barrier()

            hvec = jnp.full((16,), hh // 2, jnp.int32)

            def batch_b(b, _):
                gw = pltpu.async_copy(wout.at[tid, b], wb, sem1)
                gr = pltpu.async_copy(feat.at[hh].at[idx_s.at[b]], rows, sem2)
                gw.wait()
                gr.wait()

                def row_b(j, _):
                    wrow = wb[j, :]
                    ws = wrow.at[hvec].get(mode="promise_in_bounds")
                    for k2 in range(FW // 16):
                        sl = slice(k2 * 16, (k2 + 1) * 16)
                        rows[j, sl] = rows[j, sl] * ws
                    return 0
                lax.fori_loop(0, BB, row_b, 0)
                pltpu.sync_copy(rows, acc.at[idx_d.at[b]], add=True)
                return 0
            lax.fori_loop(0, NB, batch_b, 0)
            plsc.subcore_barrier()

            def dc(i, _):
                base = sid * RPT + i * BB
                pltpu.sync_copy(acc.at[pl.ds(base, BB)], rows)
                pltpu.sync_copy(rows, parts.at[cid, hh].at[pl.ds(base, BB)])
                return 0
            lax.fori_loop(0, NZ, dc, 0)
            plsc.subcore_barrier()
            return 0
        lax.fori_loop(0, 2 * H, head, 0)

    return edge_kernel


def _feat1_k(x_ref, w_ref, o_ref):
    hh = pl.program_id(0)
    mm = jnp.dot(x_ref[...], w_ref[0], preferred_element_type=jnp.float32)
    col = lax.broadcasted_iota(jnp.int32, (BM, FW - FH), 1)
    pad = jnp.where((hh % 2 == 0) & (col == 0), 1.0, 0.0)
    o_ref[...] = jnp.concatenate([mm, pad], axis=1)[None]


def _feat1(xp, w1h):
    return pl.pallas_call(
        _feat1_k,
        grid=(2 * HEADS, NPAD // BM),
        in_specs=[
            pl.BlockSpec((BM, F), lambda h, i: (i, 0)),
            pl.BlockSpec((1, F, FH), lambda h, i: (h, 0, 0)),
        ],
        out_specs=pl.BlockSpec((1, BM, FW), lambda h, i: (h, i, 0)),
        out_shape=jax.ShapeDtypeStruct((2 * HEADS, NPAD, FW), jnp.float32),
    )(xp, w1h)


def _alpha_k(x_ref, ws_ref, wd_ref, os_ref, od_ref):
    os_ref[...] = jnp.dot(x_ref[...], ws_ref[...],
                          preferred_element_type=jnp.float32)
    od_ref[...] = jnp.dot(x_ref[...], wd_ref[...],
                          preferred_element_type=jnp.float32)


def _alphas(xp, wsp, wdp):
    return pl.pallas_call(
        _alpha_k,
        grid=(NPAD // BM,),
        in_specs=[
            pl.BlockSpec((BM, F), lambda i: (i, 0)),
            pl.BlockSpec((F, 16), lambda i: (0, 0)),
            pl.BlockSpec((F, 16), lambda i: (0, 0)),
        ],
        out_specs=[
            pl.BlockSpec((BM, 16), lambda i: (i, 0)),
            pl.BlockSpec((BM, 16), lambda i: (i, 0)),
        ],
        out_shape=[
            jax.ShapeDtypeStruct((NPAD, 16), jnp.float32),
            jax.ShapeDtypeStruct((NPAD, 16), jnp.float32),
        ],
    )(xp, wsp, wdp)


def _comb1_k(p_ref, b1_ref, w2_ref, a2s_ref, a2d_ref,
             of_ref, os_ref, od_ref):
    acc = jnp.zeros((BM, F), jnp.float32)
    for h in range(HEADS):
        e0 = p_ref[0, 2 * h] + p_ref[1, 2 * h]
        e1 = p_ref[0, 2 * h + 1] + p_ref[1, 2 * h + 1]
        den = e0[:, FH:FH + 1] + 1e-16
        v = jnp.concatenate([e0[:, :FH], e1[:, :FH]], axis=1) / den
        v = v + b1_ref[0, h]
        v = jnp.where(v > 0, v, jnp.exp(v) - 1.0)
        acc = acc + jnp.dot(v, w2_ref[h], preferred_element_type=jnp.float32)
    ones = jnp.ones((BM, 1), jnp.float32)
    z15 = jnp.zeros((BM, FW - FH - 1), jnp.float32)
    z16 = jnp.zeros((BM, FW - FH), jnp.float32)
    f_even = jnp.concatenate([acc[:, :FH], ones, z15], axis=1)
    f_odd = jnp.concatenate([acc[:, FH:], z16], axis=1)
    of_ref[...] = jnp.stack([f_even, f_odd], axis=0)
    s = jnp.sum(acc * a2s_ref[...], axis=1, keepdims=True)
    d = jnp.sum(acc * a2d_ref[...], axis=1, keepdims=True)
    za = jnp.zeros((BM, 15), jnp.float32)
    os_ref[...] = jnp.concatenate([s, za], axis=1)
    od_ref[...] = jnp.concatenate([d, za], axis=1)


def _comb1(parts1, b1r, w2r, a2s, a2d):
    return pl.pallas_call(
        _comb1_k,
        grid=(NPAD // BM,),
        in_specs=[
            pl.BlockSpec((NC, 2 * HEADS, BM, FW), lambda i: (0, 0, i, 0)),
            pl.BlockSpec((1, HEADS, F), lambda i: (0, 0, 0)),
            pl.BlockSpec((HEADS, F, F), lambda i: (0, 0, 0)),
            pl.BlockSpec((1, F), lambda i: (0, 0)),
            pl.BlockSpec((1, F), lambda i: (0, 0)),
        ],
        out_specs=[
            pl.BlockSpec((2, BM, FW), lambda i: (0, i, 0)),
            pl.BlockSpec((BM, 16), lambda i: (i, 0)),
            pl.BlockSpec((BM, 16), lambda i: (i, 0)),
        ],
        out_shape=[
            jax.ShapeDtypeStruct((2, NPAD, FW), jnp.float32),
            jax.ShapeDtypeStruct((NPAD, 16), jnp.float32),
            jax.ShapeDtypeStruct((NPAD, 16), jnp.float32),
        ],
    )(parts1, b1r, w2r, a2s, a2d)


def _pool_k(p_ref, b2_ref, o_ref):
    i = pl.program_id(0)
    e0 = p_ref[0, 0] + p_ref[1, 0]
    e1 = p_ref[0, 1] + p_ref[1, 1]
    den = e0[:, FH:FH + 1] + 1e-16
    v = jnp.concatenate([e0[:, :FH], e1[:, :FH]], axis=1) / den
    v = jnp.maximum(v + b2_ref[...], 0.0)
    rid = i * BM + lax.broadcasted_iota(jnp.int32, (BM, 1), 0)
    v = jnp.where(rid < N, v, 0.0)
    m = jnp.max(v.reshape(BM // 8, 8, F), axis=0)

    @pl.when(i == 0)
    def _():
        o_ref[...] = m

    @pl.when(i > 0)
    def _():
        o_ref[...] = jnp.maximum(o_ref[...], m)


def _pool(parts2, b2r):
    return pl.pallas_call(
        _pool_k,
        grid=(NPAD // BM,),
        in_specs=[
            pl.BlockSpec((NC, 2, BM, FW), lambda i: (0, 0, i, 0)),
            pl.BlockSpec((1, F), lambda i: (0, 0)),
        ],
        out_specs=pl.BlockSpec((8, F), lambda i: (0, 0)),
        out_shape=jax.ShapeDtypeStruct((8, F), jnp.float32),
    )(parts2, b2r)


def _mlp_k(g_ref, w0_ref, b0_ref, w1_ref, b1_ref, w2_ref, b2_ref,
           w3_ref, b3_ref, o_ref):
    g = jnp.max(g_ref[...], axis=0, keepdims=True)
    g = jnp.maximum(jnp.dot(g, w0_ref[...],
                            preferred_element_type=jnp.float32)
                    + b0_ref[...], 0.0)
    g = jnp.maximum(jnp.dot(g, w1_ref[...],
                            preferred_element_type=jnp.float32)
                    + b1_ref[...], 0.0)
    g = jnp.maximum(jnp.dot(g, w2_ref[...],
                            preferred_element_type=jnp.float32)
                    + b2_ref[...], 0.0)
    o_ref[...] = (jnp.dot(g, w3_ref[...], preferred_element_type=jnp.float32)
                  + b3_ref[...])


def _mlp(g8, fcg_w, fcg_b, fc1_w, fc1_b, fc2_w, fc2_b, out_w, out_b):
    return pl.pallas_call(
        _mlp_k,
        out_shape=jax.ShapeDtypeStruct((1, 128), jnp.float32),
    )(g8, fcg_w, fcg_b.reshape(1, -1), fc1_w, fc1_b.reshape(1, -1),
      fc2_w, fc2_b.reshape(1, -1), out_w, out_b.reshape(1, -1))


def kernel(x, edge_index, W1, a1_src, a1_dst, b1, W2, a2_src, a2_dst, b2,
           fcg_w, fcg_b, fc1_w, fc1_b, fc2_w, fc2_b, out_w, out_b):
    loop = jnp.arange(N, dtype=edge_index.dtype)
    ndum = EPAD - E - N
    dum = N + (jnp.arange(ndum, dtype=jnp.int32) % (NPAD - N))
    src = jnp.concatenate([edge_index[0], loop, dum])
    dst = jnp.concatenate([edge_index[1], loop, dum])
    srcs = src.reshape(NTILES, NB, BB)
    dsts = dst.reshape(NTILES, NB, BB)

    xp = jnp.pad(x, ((0, NPAD - N), (0, 0)))

    w1r = W1.reshape(F, HEADS, F).transpose(1, 0, 2)
    w1h = w1r.reshape(HEADS, F, 2, FH).transpose(0, 2, 1, 3)
    w1h = w1h.reshape(2 * HEADS, F, FH)
    ws1 = jnp.einsum("fhc,hc->fh", W1.reshape(F, HEADS, F), a1_src)
    wd1 = jnp.einsum("fhc,hc->fh", W1.reshape(F, HEADS, F), a1_dst)
    wsp = jnp.pad(ws1, ((0, 0), (0, 6)))
    wdp = jnp.pad(wd1, ((0, 0), (0, 6)))
    w2r = W2.reshape(HEADS, F, F)
    b1r = b1.reshape(1, HEADS, F)
    b2r = b2.reshape(1, F)

    feat1 = _feat1(xp, w1h)
    a_s1, a_d1 = _alphas(xp, wsp, wdp)
    parts1, _ = _make_edge_sc(HEADS)(feat1, a_s1, a_d1, srcs, dsts)

    feat2, a_s2, a_d2 = _comb1(parts1, b1r, w2r, a2_src, a2_dst)
    parts2, _ = _make_edge_sc(1)(feat2, a_s2, a_d2, srcs, dsts)

    g8 = _pool(parts2, b2r)
    return _mlp(g8, fcg_w, fcg_b, fc1_w, fc1_b, fc2_w, fc2_b, out_w, out_b)


# speedup vs baseline: 10.5629x; 1.3150x over previous
import functools

import jax
import jax.numpy as jnp
from jax import lax
from jax.experimental import pallas as pl
from jax.experimental.pallas import tpu as pltpu
from jax.experimental.pallas import tpu_sc as plsc

N = 10000
E = 160000
F = 128
HEADS = 10
NPAD = 10240
NC = 2
NS = 16
NTILES = NC * NS
BB = 128
NB = 42
EPAD = NTILES * NB * BB
FH = 64
FW = 80
BM = 256
RPT = NPAD // NS
NZ = RPT // BB


def _make_edge_sc(H):
    mesh = plsc.VectorSubcoreMesh(core_axis_name="c", subcore_axis_name="s")

    @functools.partial(
        pl.kernel,
        out_type=(
            jax.ShapeDtypeStruct((NC, 2 * H, NPAD, FW), jnp.float32),
            jax.ShapeDtypeStruct((NTILES, NB, BB, 16), jnp.float32),
        ),
        mesh=mesh,
        compiler_params=pltpu.CompilerParams(use_tc_tiling_on_sc=False),
        scratch_types=[
            pltpu.VMEM((NB, BB), jnp.int32),
            pltpu.VMEM((NB, BB), jnp.int32),
            pltpu.VMEM((BB, 16), jnp.float32),
            pltpu.VMEM((BB, 16), jnp.float32),
            pltpu.VMEM((BB, 16), jnp.float32),
            pltpu.VMEM((BB, 16), jnp.float32),
            pltpu.VMEM((BB, FW), jnp.float32),
            pltpu.VMEM((BB, FW), jnp.float32),
            pltpu.VMEM((BB, FW), jnp.float32),
            pltpu.VMEM_SHARED((NPAD, FW), jnp.float32),
            pltpu.SemaphoreType.DMA,
            pltpu.SemaphoreType.DMA,
            pltpu.SemaphoreType.DMA,
            pltpu.SemaphoreType.DMA,
        ],
    )
    def edge_kernel(feat, a_s, a_d, srcs, dsts, parts, wout,
                    idx_s, idx_d, ar, ad, wb0, wb1, rows0, rows1, zbuf, acc,
                    semr0, semr1, semw0, semw1):
        cid = lax.axis_index("c")
        sid = lax.axis_index("s")
        tid = cid * NS + sid
        fzero = jnp.zeros((16,), jnp.float32)

        def zb(j, _):
            for k2 in range(FW // 16):
                zbuf[j, k2 * 16:(k2 + 1) * 16] = fzero
            return 0
        lax.fori_loop(0, BB, zb, 0)

        pltpu.sync_copy(srcs.at[tid], idx_s)
        pltpu.sync_copy(dsts.at[tid], idx_d)

        def batch_a(b, _):
            ga = pltpu.async_copy(a_s.at[idx_s.at[b]], ar, semr0)
            gb = pltpu.async_copy(a_d.at[idx_d.at[b]], ad, semr1)
            ga.wait()
            gb.wait()

            def row_a(j, _):
                e = ar[j, :] + ad[j, :]
                e = jnp.where(e >= 0.0, e, e * 0.2)
                wb0[j, :] = jnp.exp(e)
                return 0
            lax.fori_loop(0, BB, row_a, 0, unroll=4)
            pltpu.sync_copy(wb0, wout.at[tid, b])
            return 0
        lax.fori_loop(0, NB, batch_a, 0)

        def head(hh, _):
            def zc(i, _):
                pltpu.sync_copy(zbuf, acc.at[pl.ds(sid * RPT + i * BB, BB)])
                return 0
            lax.fori_loop(0, NZ, zc, 0)
            plsc.subcore_barrier()

            hvec = jnp.full((16,), hh // 2, jnp.int32)

            def start(b, rbuf, wbuf, sr, sw):
                pltpu.async_copy(wout.at[tid, b], wbuf, sw)
                pltpu.async_copy(feat.at[hh].at[idx_s.at[b]], rbuf, sr)

            def finish(b, rbuf, wbuf, sr, sw):
                pltpu.make_async_copy(wout.at[tid, b], wbuf, sw).wait()
                pltpu.make_async_copy(
                    feat.at[hh].at[idx_s.at[b]], rbuf, sr).wait()

                def row_b(j, _):
                    wrow = wbuf[j, :]
                    ws = wrow.at[hvec].get(mode="promise_in_bounds")
                    for k2 in range(FW // 16):
                        sl = slice(k2 * 16, (k2 + 1) * 16)
                        rbuf[j, sl] = rbuf[j, sl] * ws
                    return 0
                lax.fori_loop(0, BB, row_b, 0, unroll=4)
                pltpu.sync_copy(rbuf, acc.at[idx_d.at[b]], add=True)

            start(0, rows0, wb0, semr0, semw0)

            def pair(bb, _):
                b0 = 2 * bb
                start(b0 + 1, rows1, wb1, semr1, semw1)
                finish(b0, rows0, wb0, semr0, semw0)

                @pl.when(b0 + 2 < NB)
                def _():
                    start(b0 + 2, rows0, wb0, semr0, semw0)
                finish(b0 + 1, rows1, wb1, semr1, semw1)
                return 0
            lax.fori_loop(0, NB // 2, pair, 0)
            plsc.subcore_barrier()

            def dc(i, _):
                base = sid * RPT + i * BB
                pltpu.sync_copy(acc.at[pl.ds(base, BB)], rows0)
                pltpu.sync_copy(rows0, parts.at[cid, hh].at[pl.ds(base, BB)])
                return 0
            lax.fori_loop(0, NZ, dc, 0)
            plsc.subcore_barrier()
            return 0
        lax.fori_loop(0, 2 * H, head, 0)

    return edge_kernel


def _feat1_k(x_ref, w_ref, o_ref):
    hh = pl.program_id(0)
    mm = jnp.dot(x_ref[...], w_ref[0], preferred_element_type=jnp.float32)
    col = lax.broadcasted_iota(jnp.int32, (BM, FW - FH), 1)
    pad = jnp.where((hh % 2 == 0) & (col == 0), 1.0, 0.0)
    o_ref[...] = jnp.concatenate([mm, pad], axis=1)[None]


def _feat1(xp, w1h):
    return pl.pallas_call(
        _feat1_k,
        grid=(2 * HEADS, NPAD // BM),
        in_specs=[
            pl.BlockSpec((BM, F), lambda h, i: (i, 0)),
            pl.BlockSpec((1, F, FH), lambda h, i: (h, 0, 0)),
        ],
        out_specs=pl.BlockSpec((1, BM, FW), lambda h, i: (h, i, 0)),
        out_shape=jax.ShapeDtypeStruct((2 * HEADS, NPAD, FW), jnp.float32),
    )(xp, w1h)


def _alpha_k(x_ref, ws_ref, wd_ref, os_ref, od_ref):
    os_ref[...] = jnp.dot(x_ref[...], ws_ref[...],
                          preferred_element_type=jnp.float32)
    od_ref[...] = jnp.dot(x_ref[...], wd_ref[...],
                          preferred_element_type=jnp.float32)


def _alphas(xp, wsp, wdp):
    return pl.pallas_call(
        _alpha_k,
        grid=(NPAD // BM,),
        in_specs=[
            pl.BlockSpec((BM, F), lambda i: (i, 0)),
            pl.BlockSpec((F, 16), lambda i: (0, 0)),
            pl.BlockSpec((F, 16), lambda i: (0, 0)),
        ],
        out_specs=[
            pl.BlockSpec((BM, 16), lambda i: (i, 0)),
            pl.BlockSpec((BM, 16), lambda i: (i, 0)),
        ],
        out_shape=[
            jax.ShapeDtypeStruct((NPAD, 16), jnp.float32),
            jax.ShapeDtypeStruct((NPAD, 16), jnp.float32),
        ],
    )(xp, wsp, wdp)


def _comb1_k(p_ref, b1_ref, w2_ref, a2s_ref, a2d_ref,
             of_ref, os_ref, od_ref):
    acc = jnp.zeros((BM, F), jnp.float32)
    for h in range(HEADS):
        e0 = p_ref[0, 2 * h] + p_ref[1, 2 * h]
        e1 = p_ref[0, 2 * h + 1] + p_ref[1, 2 * h + 1]
        den = e0[:, FH:FH + 1] + 1e-16
        v = jnp.concatenate([e0[:, :FH], e1[:, :FH]], axis=1) / den
        v = v + b1_ref[0, h]
        v = jnp.where(v > 0, v, jnp.exp(v) - 1.0)
        acc = acc + jnp.dot(v, w2_ref[h], preferred_element_type=jnp.float32)
    ones = jnp.ones((BM, 1), jnp.float32)
    z15 = jnp.zeros((BM, FW - FH - 1), jnp.float32)
    z16 = jnp.zeros((BM, FW - FH), jnp.float32)
    f_even = jnp.concatenate([acc[:, :FH], ones, z15], axis=1)
    f_odd = jnp.concatenate([acc[:, FH:], z16], axis=1)
    of_ref[...] = jnp.stack([f_even, f_odd], axis=0)
    s = jnp.sum(acc * a2s_ref[...], axis=1, keepdims=True)
    d = jnp.sum(acc * a2d_ref[...], axis=1, keepdims=True)
    za = jnp.zeros((BM, 15), jnp.float32)
    os_ref[...] = jnp.concatenate([s, za], axis=1)
    od_ref[...] = jnp.concatenate([d, za], axis=1)


def _comb1(parts1, b1r, w2r, a2s, a2d):
    return pl.pallas_call(
        _comb1_k,
        grid=(NPAD // BM,),
        in_specs=[
            pl.BlockSpec((NC, 2 * HEADS, BM, FW), lambda i: (0, 0, i, 0)),
            pl.BlockSpec((1, HEADS, F), lambda i: (0, 0, 0)),
            pl.BlockSpec((HEADS, F, F), lambda i: (0, 0, 0)),
            pl.BlockSpec((1, F), lambda i: (0, 0)),
            pl.BlockSpec((1, F), lambda i: (0, 0)),
        ],
        out_specs=[
            pl.BlockSpec((2, BM, FW), lambda i: (0, i, 0)),
            pl.BlockSpec((BM, 16), lambda i: (i, 0)),
            pl.BlockSpec((BM, 16), lambda i: (i, 0)),
        ],
        out_shape=[
            jax.ShapeDtypeStruct((2, NPAD, FW), jnp.float32),
            jax.ShapeDtypeStruct((NPAD, 16), jnp.float32),
            jax.ShapeDtypeStruct((NPAD, 16), jnp.float32),
        ],
    )(parts1, b1r, w2r, a2s, a2d)


def _pool_k(p_ref, b2_ref, o_ref):
    i = pl.program_id(0)
    e0 = p_ref[0, 0] + p_ref[1, 0]
    e1 = p_ref[0, 1] + p_ref[1, 1]
    den = e0[:, FH:FH + 1] + 1e-16
    v = jnp.concatenate([e0[:, :FH], e1[:, :FH]], axis=1) / den
    v = jnp.maximum(v + b2_ref[...], 0.0)
    rid = i * BM + lax.broadcasted_iota(jnp.int32, (BM, 1), 0)
    v = jnp.where(rid < N, v, 0.0)
    m = jnp.max(v.reshape(BM // 8, 8, F), axis=0)

    @pl.when(i == 0)
    def _():
        o_ref[...] = m

    @pl.when(i > 0)
    def _():
        o_ref[...] = jnp.maximum(o_ref[...], m)


def _pool(parts2, b2r):
    return pl.pallas_call(
        _pool_k,
        grid=(NPAD // BM,),
        in_specs=[
            pl.BlockSpec((NC, 2, BM, FW), lambda i: (0, 0, i, 0)),
            pl.BlockSpec((1, F), lambda i: (0, 0)),
        ],
        out_specs=pl.BlockSpec((8, F), lambda i: (0, 0)),
        out_shape=jax.ShapeDtypeStruct((8, F), jnp.float32),
    )(parts2, b2r)


def _mlp_k(g_ref, w0_ref, b0_ref, w1_ref, b1_ref, w2_ref, b2_ref,
           w3_ref, b3_ref, o_ref):
    g = jnp.max(g_ref[...], axis=0, keepdims=True)
    g = jnp.maximum(jnp.dot(g, w0_ref[...],
                            preferred_element_type=jnp.float32)
                    + b0_ref[...], 0.0)
    g = jnp.maximum(jnp.dot(g, w1_ref[...],
                            preferred_element_type=jnp.float32)
                    + b1_ref[...], 0.0)
    g = jnp.maximum(jnp.dot(g, w2_ref[...],
                            preferred_element_type=jnp.float32)
                    + b2_ref[...], 0.0)
    o_ref[...] = (jnp.dot(g, w3_ref[...], preferred_element_type=jnp.float32)
                  + b3_ref[...])


def _mlp(g8, fcg_w, fcg_b, fc1_w, fc1_b, fc2_w, fc2_b, out_w, out_b):
    return pl.pallas_call(
        _mlp_k,
        out_shape=jax.ShapeDtypeStruct((1, 128), jnp.float32),
    )(g8, fcg_w, fcg_b.reshape(1, -1), fc1_w, fc1_b.reshape(1, -1),
      fc2_w, fc2_b.reshape(1, -1), out_w, out_b.reshape(1, -1))


def kernel(x, edge_index, W1, a1_src, a1_dst, b1, W2, a2_src, a2_dst, b2,
           fcg_w, fcg_b, fc1_w, fc1_b, fc2_w, fc2_b, out_w, out_b):
    loop = jnp.arange(N, dtype=edge_index.dtype)
    ndum = EPAD - E - N
    dum = N + (jnp.arange(ndum, dtype=jnp.int32) % (NPAD - N))
    src = jnp.concatenate([edge_index[0], loop, dum])
    dst = jnp.concatenate([edge_index[1], loop, dum])
    srcs = src.reshape(NTILES, NB, BB)
    dsts = dst.reshape(NTILES, NB, BB)

    xp = jnp.pad(x, ((0, NPAD - N), (0, 0)))

    w1r = W1.reshape(F, HEADS, F).transpose(1, 0, 2)
    w1h = w1r.reshape(HEADS, F, 2, FH).transpose(0, 2, 1, 3)
    w1h = w1h.reshape(2 * HEADS, F, FH)
    ws1 = jnp.einsum("fhc,hc->fh", W1.reshape(F, HEADS, F), a1_src)
    wd1 = jnp.einsum("fhc,hc->fh", W1.reshape(F, HEADS, F), a1_dst)
    wsp = jnp.pad(ws1, ((0, 0), (0, 6)))
    wdp = jnp.pad(wd1, ((0, 0), (0, 6)))
    w2r = W2.reshape(HEADS, F, F)
    b1r = b1.reshape(1, HEADS, F)
    b2r = b2.reshape(1, F)

    feat1 = _feat1(xp, w1h)
    a_s1, a_d1 = _alphas(xp, wsp, wdp)
    parts1, _ = _make_edge_sc(HEADS)(feat1, a_s1, a_d1, srcs, dsts)

    feat2, a_s2, a_d2 = _comb1(parts1, b1r, w2r, a2_src, a2_dst)
    parts2, _ = _make_edge_sc(1)(feat2, a_s2, a_d2, srcs, dsts)

    g8 = _pool(parts2, b2r)
    return _mlp(g8, fcg_w, fcg_b, fc1_w, fc1_b, fc2_w, fc2_b, out_w, out_b)


# speedup vs baseline: 11.7507x; 1.1125x over previous
import functools

import jax
import jax.numpy as jnp
from jax import lax
from jax.experimental import pallas as pl
from jax.experimental.pallas import tpu as pltpu
from jax.experimental.pallas import tpu_sc as plsc

N = 10000
E = 160000
F = 128
HEADS = 10
NPAD = 10240
NC = 2
NS = 16
NTILES = NC * NS
BB = 128
NB = 42
EPAD = NTILES * NB * BB
FH = 64
FW = 80
BM = 256
RPT = NPAD // NS
NZ = RPT // BB


def _make_edge_sc(H):
    mesh = plsc.VectorSubcoreMesh(core_axis_name="c", subcore_axis_name="s")

    @functools.partial(
        pl.kernel,
        out_type=(
            jax.ShapeDtypeStruct((NC, 2 * H, NPAD, FW), jnp.float32),
            jax.ShapeDtypeStruct((NTILES, NB, BB, 16), jnp.float32),
        ),
        mesh=mesh,
        compiler_params=pltpu.CompilerParams(use_tc_tiling_on_sc=False),
        scratch_types=[
            pltpu.VMEM((NB, BB), jnp.int32),
            pltpu.VMEM((NB, BB), jnp.int32),
            pltpu.VMEM((BB, 16), jnp.float32),
            pltpu.VMEM((BB, 16), jnp.float32),
            pltpu.VMEM((BB, 16), jnp.float32),
            pltpu.VMEM((BB, 16), jnp.float32),
            pltpu.VMEM((BB, FW), jnp.float32),
            pltpu.VMEM((BB, FW), jnp.float32),
            pltpu.VMEM((BB, FW), jnp.float32),
            pltpu.VMEM_SHARED((NPAD, FW), jnp.float32),
            pltpu.SemaphoreType.DMA,
            pltpu.SemaphoreType.DMA,
            pltpu.SemaphoreType.DMA,
            pltpu.SemaphoreType.DMA,
            pltpu.SemaphoreType.DMA,
            pltpu.SemaphoreType.DMA,
        ],
    )
    def edge_kernel(feat, a_s, a_d, srcs, dsts, parts, wout,
                    idx_s, idx_d, ar, ad, wb0, wb1, rows0, rows1, zbuf, acc,
                    semr0, semr1, semw0, semw1, sems0, sems1):
        cid = lax.axis_index("c")
        sid = lax.axis_index("s")
        tid = cid * NS + sid
        fzero = jnp.zeros((16,), jnp.float32)

        def zb(j, _):
            for k2 in range(FW // 16):
                zbuf[j, k2 * 16:(k2 + 1) * 16] = fzero
            return 0
        lax.fori_loop(0, BB, zb, 0)

        pltpu.sync_copy(srcs.at[tid], idx_s)
        pltpu.sync_copy(dsts.at[tid], idx_d)

        def batch_a(b, _):
            ga = pltpu.async_copy(a_s.at[idx_s.at[b]], ar, semr0)
            gb = pltpu.async_copy(a_d.at[idx_d.at[b]], ad, semr1)
            ga.wait()
            gb.wait()

            def row_a(j, _):
                e = ar[j, :] + ad[j, :]
                e = jnp.where(e >= 0.0, e, e * 0.2)
                wb0[j, :] = jnp.exp(e)
                return 0
            lax.fori_loop(0, BB, row_a, 0, unroll=4)
            pltpu.sync_copy(wb0, wout.at[tid, b])
            return 0
        lax.fori_loop(0, NB, batch_a, 0)

        def head(hh, _):
            def zc(i, _):
                pltpu.sync_copy(zbuf, acc.at[pl.ds(sid * RPT + i * BB, BB)])
                return 0
            lax.fori_loop(0, NZ, zc, 0)
            plsc.subcore_barrier()

            hvec = jnp.full((16,), hh // 2, jnp.int32)

            def start(b, rbuf, wbuf, sr, sw):
                pltpu.async_copy(wout.at[tid, b], wbuf, sw)
                pltpu.async_copy(feat.at[hh].at[idx_s.at[b]], rbuf, sr)

            def scale(b, rbuf, wbuf, sr, sw):
                pltpu.make_async_copy(wout.at[tid, b], wbuf, sw).wait()
                pltpu.make_async_copy(
                    feat.at[hh].at[idx_s.at[b]], rbuf, sr).wait()

                def row_b(j, _):
                    wrow = wbuf[j, :]
                    ws = wrow.at[hvec].get(mode="promise_in_bounds")
                    for k2 in range(FW // 16):
                        sl = slice(k2 * 16, (k2 + 1) * 16)
                        rbuf[j, sl] = rbuf[j, sl] * ws
                    return 0
                lax.fori_loop(0, BB, row_b, 0, unroll=4)

            def scat_start(b, rbuf, ss):
                pltpu.async_copy(rbuf, acc.at[idx_d.at[b]], ss, add=True)

            def scat_wait(b, rbuf, ss):
                pltpu.make_async_copy(
                    rbuf, acc.at[idx_d.at[b]], ss).wait()

            start(0, rows0, wb0, semr0, semw0)

            def pair(bb, _):
                b0 = 2 * bb

                @pl.when(bb > 0)
                def _():
                    scat_wait(b0 - 1, rows1, sems1)
                start(b0 + 1, rows1, wb1, semr1, semw1)
                scale(b0, rows0, wb0, semr0, semw0)
                scat_start(b0, rows0, sems0)
                scale(b0 + 1, rows1, wb1, semr1, semw1)
                scat_wait(b0, rows0, sems0)

                @pl.when(b0 + 2 < NB)
                def _():
                    start(b0 + 2, rows0, wb0, semr0, semw0)
                scat_start(b0 + 1, rows1, sems1)
                return 0
            lax.fori_loop(0, NB // 2, pair, 0)
            scat_wait(NB - 1, rows1, sems1)
            plsc.subcore_barrier()

            def dc(i, _):
                base = sid * RPT + i * BB
                pltpu.sync_copy(acc.at[pl.ds(base, BB)], rows0)
                pltpu.sync_copy(rows0, parts.at[cid, hh].at[pl.ds(base, BB)])
                return 0
            lax.fori_loop(0, NZ, dc, 0)
            plsc.subcore_barrier()
            return 0
        lax.fori_loop(0, 2 * H, head, 0)

    return edge_kernel


def _feat1_k(x_ref, w_ref, o_ref):
    mm = jnp.dot(x_ref[...], w_ref[...], preferred_element_type=jnp.float32)
    col = lax.broadcasted_iota(jnp.int32, (BM, FW - FH), 1)
    pad1 = jnp.where(col == 0, 1.0, 0.0)
    pad0 = jnp.zeros((BM, FW - FH), jnp.float32)
    for hh in range(2 * HEADS):
        pad = pad1 if hh % 2 == 0 else pad0
        o_ref[hh] = jnp.concatenate(
            [mm[:, hh * FH:(hh + 1) * FH], pad], axis=1)


def _feat1(xp, w1c):
    return pl.pallas_call(
        _feat1_k,
        grid=(NPAD // BM,),
        in_specs=[
            pl.BlockSpec((BM, F), lambda i: (i, 0)),
            pl.BlockSpec((F, 2 * HEADS * FH), lambda i: (0, 0)),
        ],
        out_specs=pl.BlockSpec((2 * HEADS, BM, FW), lambda i: (0, i, 0)),
        out_shape=jax.ShapeDtypeStruct((2 * HEADS, NPAD, FW), jnp.float32),
    )(xp, w1c)


def _alpha_k(x_ref, ws_ref, wd_ref, os_ref, od_ref):
    os_ref[...] = jnp.dot(x_ref[...], ws_ref[...],
                          preferred_element_type=jnp.float32)
    od_ref[...] = jnp.dot(x_ref[...], wd_ref[...],
                          preferred_element_type=jnp.float32)


def _alphas(xp, wsp, wdp):
    return pl.pallas_call(
        _alpha_k,
        grid=(NPAD // BM,),
        in_specs=[
            pl.BlockSpec((BM, F), lambda i: (i, 0)),
            pl.BlockSpec((F, 16), lambda i: (0, 0)),
            pl.BlockSpec((F, 16), lambda i: (0, 0)),
        ],
        out_specs=[
            pl.BlockSpec((BM, 16), lambda i: (i, 0)),
            pl.BlockSpec((BM, 16), lambda i: (i, 0)),
        ],
        out_shape=[
            jax.ShapeDtypeStruct((NPAD, 16), jnp.float32),
            jax.ShapeDtypeStruct((NPAD, 16), jnp.float32),
        ],
    )(xp, wsp, wdp)


def _comb1_k(p_ref, b1_ref, w2_ref, a2s_ref, a2d_ref,
             of_ref, os_ref, od_ref):
    acc = jnp.zeros((BM, F), jnp.float32)
    for h in range(HEADS):
        e0 = p_ref[0, 2 * h] + p_ref[1, 2 * h]
        e1 = p_ref[0, 2 * h + 1] + p_ref[1, 2 * h + 1]
        den = e0[:, FH:FH + 1] + 1e-16
        v = jnp.concatenate([e0[:, :FH], e1[:, :FH]], axis=1) / den
        v = v + b1_ref[0, h]
        v = jnp.where(v > 0, v, jnp.exp(v) - 1.0)
        acc = acc + jnp.dot(v, w2_ref[h], preferred_element_type=jnp.float32)
    ones = jnp.ones((BM, 1), jnp.float32)
    z15 = jnp.zeros((BM, FW - FH - 1), jnp.float32)
    z16 = jnp.zeros((BM, FW - FH), jnp.float32)
    f_even = jnp.concatenate([acc[:, :FH], ones, z15], axis=1)
    f_odd = jnp.concatenate([acc[:, FH:], z16], axis=1)
    of_ref[...] = jnp.stack([f_even, f_odd], axis=0)
    s = jnp.sum(acc * a2s_ref[...], axis=1, keepdims=True)
    d = jnp.sum(acc * a2d_ref[...], axis=1, keepdims=True)
    za = jnp.zeros((BM, 15), jnp.float32)
    os_ref[...] = jnp.concatenate([s, za], axis=1)
    od_ref[...] = jnp.concatenate([d, za], axis=1)


def _comb1(parts1, b1r, w2r, a2s, a2d):
    return pl.pallas_call(
        _comb1_k,
        grid=(NPAD // BM,),
        in_specs=[
            pl.BlockSpec((NC, 2 * HEADS, BM, FW), lambda i: (0, 0, i, 0)),
            pl.BlockSpec((1, HEADS, F), lambda i: (0, 0, 0)),
            pl.BlockSpec((HEADS, F, F), lambda i: (0, 0, 0)),
            pl.BlockSpec((1, F), lambda i: (0, 0)),
            pl.BlockSpec((1, F), lambda i: (0, 0)),
        ],
        out_specs=[
            pl.BlockSpec((2, BM, FW), lambda i: (0, i, 0)),
            pl.BlockSpec((BM, 16), lambda i: (i, 0)),
            pl.BlockSpec((BM, 16), lambda i: (i, 0)),
        ],
        out_shape=[
            jax.ShapeDtypeStruct((2, NPAD, FW), jnp.float32),
            jax.ShapeDtypeStruct((NPAD, 16), jnp.float32),
            jax.ShapeDtypeStruct((NPAD, 16), jnp.float32),
        ],
    )(parts1, b1r, w2r, a2s, a2d)


def _pool_k(p_ref, b2_ref, o_ref):
    i = pl.program_id(0)
    e0 = p_ref[0, 0] + p_ref[1, 0]
    e1 = p_ref[0, 1] + p_ref[1, 1]
    den = e0[:, FH:FH + 1] + 1e-16
    v = jnp.concatenate([e0[:, :FH], e1[:, :FH]], axis=1) / den
    v = jnp.maximum(v + b2_ref[...], 0.0)
    rid = i * BM + lax.broadcasted_iota(jnp.int32, (BM, 1), 0)
    v = jnp.where(rid < N, v, 0.0)
    m = jnp.max(v.reshape(BM // 8, 8, F), axis=0)

    @pl.when(i == 0)
    def _():
        o_ref[...] = m

    @pl.when(i > 0)
    def _():
        o_ref[...] = jnp.maximum(o_ref[...], m)


def _pool(parts2, b2r):
    return pl.pallas_call(
        _pool_k,
        grid=(NPAD // BM,),
        in_specs=[
            pl.BlockSpec((NC, 2, BM, FW), lambda i: (0, 0, i, 0)),
            pl.BlockSpec((1, F), lambda i: (0, 0)),
        ],
        out_specs=pl.BlockSpec((8, F), lambda i: (0, 0)),
        out_shape=jax.ShapeDtypeStruct((8, F), jnp.float32),
    )(parts2, b2r)


def _mlp_k(g_ref, w0_ref, b0_ref, w1_ref, b1_ref, w2_ref, b2_ref,
           w3_ref, b3_ref, o_ref):
    g = jnp.max(g_ref[...], axis=0, keepdims=True)
    g = jnp.maximum(jnp.dot(g, w0_ref[...],
                            preferred_element_type=jnp.float32)
                    + b0_ref[...], 0.0)
    g = jnp.maximum(jnp.dot(g, w1_ref[...],
                            preferred_element_type=jnp.float32)
                    + b1_ref[...], 0.0)
    g = jnp.maximum(jnp.dot(g, w2_ref[...],
                            preferred_element_type=jnp.float32)
                    + b2_ref[...], 0.0)
    o_ref[...] = (jnp.dot(g, w3_ref[...], preferred_element_type=jnp.float32)
                  + b3_ref[...])


def _mlp(g8, fcg_w, fcg_b, fc1_w, fc1_b, fc2_w, fc2_b, out_w, out_b):
    return pl.pallas_call(
        _mlp_k,
        out_shape=jax.ShapeDtypeStruct((1, 128), jnp.float32),
    )(g8, fcg_w, fcg_b.reshape(1, -1), fc1_w, fc1_b.reshape(1, -1),
      fc2_w, fc2_b.reshape(1, -1), out_w, out_b.reshape(1, -1))


def kernel(x, edge_index, W1, a1_src, a1_dst, b1, W2, a2_src, a2_dst, b2,
           fcg_w, fcg_b, fc1_w, fc1_b, fc2_w, fc2_b, out_w, out_b):
    loop = jnp.arange(N, dtype=edge_index.dtype)
    ndum = EPAD - E - N
    dum = N + (jnp.arange(ndum, dtype=jnp.int32) % (NPAD - N))
    src = jnp.concatenate([edge_index[0], loop, dum])
    dst = jnp.concatenate([edge_index[1], loop, dum])
    srcs = src.reshape(NTILES, NB, BB)
    dsts = dst.reshape(NTILES, NB, BB)

    xp = jnp.pad(x, ((0, NPAD - N), (0, 0)))

    w1r = W1.reshape(F, HEADS, F).transpose(1, 0, 2)
    w1h = w1r.reshape(HEADS, F, 2, FH).transpose(0, 2, 1, 3)
    w1c = w1h.reshape(2 * HEADS, F, FH).transpose(1, 0, 2)
    w1c = w1c.reshape(F, 2 * HEADS * FH)
    ws1 = jnp.einsum("fhc,hc->fh", W1.reshape(F, HEADS, F), a1_src)
    wd1 = jnp.einsum("fhc,hc->fh", W1.reshape(F, HEADS, F), a1_dst)
    wsp = jnp.pad(ws1, ((0, 0), (0, 6)))
    wdp = jnp.pad(wd1, ((0, 0), (0, 6)))
    w2r = W2.reshape(HEADS, F, F)
    b1r = b1.reshape(1, HEADS, F)
    b2r = b2.reshape(1, F)

    feat1 = _feat1(xp, w1c)
    a_s1, a_d1 = _alphas(xp, wsp, wdp)
    parts1, _ = _make_edge_sc(HEADS)(feat1, a_s1, a_d1, srcs, dsts)

    feat2, a_s2, a_d2 = _comb1(parts1, b1r, w2r, a2_src, a2_dst)
    parts2, _ = _make_edge_sc(1)(feat2, a_s2, a_d2, srcs, dsts)

    g8 = _pool(parts2, b2r)
    return _mlp(g8, fcg_w, fcg_b, fc1_w, fc1_b, fc2_w, fc2_b, out_w, out_b)
